# baseline (speedup 1.0000x reference)
import functools

import jax
import jax.numpy as jnp
from jax import lax
from jax.experimental import pallas as pl
from jax.experimental.pallas import tpu as pltpu
from jax.experimental.pallas import tpu_sc as plsc


def _mm_body(x_ref, w_ref, o_ref):
    o_ref[...] = jnp.dot(x_ref[...], w_ref[...],
                         preferred_element_type=jnp.float32)


def _fused_body(a_ref, b_ref, bias_ref, w_ref, o_ref):
    h = jnp.maximum(a_ref[...] + b_ref[...] + bias_ref[...], 0.0)
    o_ref[...] = jnp.dot(h, w_ref[...], preferred_element_type=jnp.float32)


def _add_body(a_ref, b_ref, bias_ref, o_ref):
    o_ref[...] = a_ref[...] + b_ref[...] + bias_ref[...]


def _tc_matmul(x, w, block_rows=2000):
    n, d_in = x.shape
    d_out = w.shape[1]
    return pl.pallas_call(
        _mm_body,
        grid=(n // block_rows,),
        in_specs=[
            pl.BlockSpec((block_rows, d_in), lambda i: (i, 0)),
            pl.BlockSpec((d_in, d_out), lambda i: (0, 0)),
        ],
        out_specs=pl.BlockSpec((block_rows, d_out), lambda i: (i, 0)),
        out_shape=jax.ShapeDtypeStruct((n, d_out), jnp.float32),
    )(x, w)


def _tc_fused(a, b, bias, w, block_rows=2000):
    n, d_in = a.shape
    d_out = w.shape[1]
    bias2 = bias.reshape(1, d_in)
    return pl.pallas_call(
        _fused_body,
        grid=(n // block_rows,),
        in_specs=[
            pl.BlockSpec((block_rows, d_in), lambda i: (i, 0)),
            pl.BlockSpec((block_rows, d_in), lambda i: (i, 0)),
            pl.BlockSpec((1, d_in), lambda i: (0, 0)),
            pl.BlockSpec((d_in, d_out), lambda i: (0, 0)),
        ],
        out_specs=pl.BlockSpec((block_rows, d_out), lambda i: (i, 0)),
        out_shape=jax.ShapeDtypeStruct((n, d_out), jnp.float32),
    )(a, b, bias2, w)


def _tc_add(a, b, bias, block_rows=2000):
    n, d = a.shape
    bias2 = bias.reshape(1, d)
    return pl.pallas_call(
        _add_body,
        grid=(n // block_rows,),
        in_specs=[
            pl.BlockSpec((block_rows, d), lambda i: (i, 0)),
            pl.BlockSpec((block_rows, d), lambda i: (i, 0)),
            pl.BlockSpec((1, d), lambda i: (0, 0)),
        ],
        out_specs=pl.BlockSpec((block_rows, d), lambda i: (i, 0)),
        out_shape=jax.ShapeDtypeStruct((n, d), jnp.float32),
    )(a, b, bias2)


_SC_K = 32
_NBUF = 4


@functools.lru_cache(maxsize=None)
def _make_sc_scatter(n_nodes, n_edges, d, q0, q1):
    info = plsc.get_sparse_core_info()
    nc, ns, lanes = info.num_cores, info.num_subcores, info.num_lanes
    k = _SC_K
    nb = _NBUF
    qmax = max(q0, q1)
    rows_per_tile = n_nodes // ns
    assert n_edges == ns * (q0 + q1) * k and q0 % nb == 0 and q1 % nb == 0
    assert n_nodes % (8 * ns) == 0 and d % lanes == 0
    assert rows_per_tile % 8 == 0

    mesh = plsc.VectorSubcoreMesh(core_axis_name="c", subcore_axis_name="s")
    _bcast_dn = lax.GatherDimensionNumbers(
        offset_dims=(), collapsed_slice_dims=(0,), start_index_map=(0,))

    @functools.partial(
        pl.kernel,
        mesh=mesh,
        compiler_params=pltpu.CompilerParams(use_tc_tiling_on_sc=False),
        out_type=jax.ShapeDtypeStruct((nc, n_nodes, d), jnp.float32),
        scratch_types=[
            pltpu.VMEM_SHARED((n_nodes, d), jnp.float32),
            pltpu.VMEM((qmax, k), jnp.int32),
            pltpu.VMEM((qmax, k), jnp.float32),
            pltpu.VMEM((nb, 2, k), jnp.int32),
            pltpu.VMEM((nb, k, d), jnp.float32),
            pltpu.SemaphoreType.DMA((nb,)),
            pltpu.SemaphoreType.DMA((nb,)),
        ],
    )
    def sc_scatter(p_hbm, sp_hbm, w_hbm, out_hbm,
                   agg, sp_v, w_v, idx_v, rows_v, gsem, ssem):
        cid = lax.axis_index("c")
        sid = lax.axis_index("s")

        my_rows = pl.ds(sid * rows_per_tile, rows_per_tile)
        nq = jnp.where(cid == 0, q0, q1)
        n_quads_t = nq // nb
        cbase = jnp.where(cid == 0, sid * q0, ns * q0 + sid * q1)
        pre_sp = pltpu.make_async_copy(
            sp_hbm.at[pl.ds(cbase, qmax)], sp_v, gsem.at[0])
        pre_w = pltpu.make_async_copy(
            w_hbm.at[pl.ds(cbase, qmax)], w_v, gsem.at[1])
        pre_sp.start()
        pre_w.start()

        zv = jnp.zeros((lanes,), jnp.float32)

        def zrow_body(r, carry):
            for j in range(d // lanes):
                rows_v[0, r, pl.ds(j * lanes, lanes)] = zv
            return carry

        lax.fori_loop(0, k, zrow_body, 0)
        for t in range(rows_per_tile // k):
            pltpu.sync_copy(
                rows_v.at[0],
                agg.at[pl.ds(sid * rows_per_tile + t * k, k)])
        rem = rows_per_tile % k
        if rem:
            pltpu.sync_copy(
                rows_v.at[0, pl.ds(0, rem)],
                agg.at[pl.ds(sid * rows_per_tile + rows_per_tile - rem, rem)])
        pre_sp.wait()
        pre_w.wait()
        plsc.subcore_barrier()

        def unpack(c, b):
            for t in range(k // lanes):
                sl = pl.ds(t * lanes, lanes)
                v = sp_v[c, sl]
                idx_v[b, 0, sl] = v & 0xFFFF
                idx_v[b, 1, sl] = (v >> 16) & 0xFFFF

        def gather(b, c):
            return pltpu.make_async_copy(
                p_hbm.at[idx_v.at[b, 0]], rows_v.at[b], gsem.at[b])

        def scatter(b, c):
            return pltpu.make_async_copy(
                rows_v.at[b], agg.at[idx_v.at[b, 1]], ssem.at[b])

        unpack(0, 0)
        unpack(1, 1)
        gather(0, 0).start()
        gather(1, 1).start()

        def quad_body(cc, carry):
            for b in range(nb):
                c = nb * cc + b
                b2 = (b + 2) % nb
                gather(b, c).wait()
                if b < 2:
                    @pl.when(cc > 0)
                    def _():
                        scatter(b2, c).wait()

                    unpack(c + 2, b2)
                    gather(b2, c + 2).start()
                else:
                    scatter(b2, c).wait()

                    @pl.when(cc < n_quads_t - 1)
                    def _():
                        unpack(c + 2, b2)
                        gather(b2, c + 2).start()

                for t in range(k // lanes):
                    wv16 = w_v[c, pl.ds(t * lanes, lanes)]
                    for l in range(lanes):
                        wb = lax.gather(
                            wv16, jnp.full((lanes, 1), l, jnp.int32),
                            _bcast_dn, (1,),
                            mode=lax.GatherScatterMode.PROMISE_IN_BOUNDS)
                        i = t * lanes + l
                        for j in range(d // lanes):
                            sl = pl.ds(j * lanes, lanes)
                            rows_v[b, i, sl] = rows_v[b, i, sl] * wb
                scatter(b, c).start(add=True)
            return carry

        lax.fori_loop(0, n_quads_t, quad_body, 0)
        scatter(2, 0).wait()
        scatter(3, 0).wait()
        plsc.subcore_barrier()
        pltpu.sync_copy(agg.at[my_rows], out_hbm.at[cid, my_rows])

    return sc_scatter


def kernel(features, g, edge_weight, W1, b1, W2, b2, W3, b3):
    n_nodes, d_in = features.shape
    n_edges = g.shape[1]
    d_h = W1.shape[1]
    n_classes = W3.shape[1]
    lanes = 16
    d_out = ((n_classes + lanes - 1) // lanes) * lanes

    W3p = jnp.pad(W3, ((0, 0), (0, d_out - n_classes)))
    b3p = jnp.pad(b3, (0, d_out - n_classes))

    info = plsc.get_sparse_core_info()
    ns, nc = info.num_subcores, info.num_cores
    nw = nc * ns
    k = _SC_K
    e_pad = -(-n_edges // (nw * _NBUF * k)) * (nw * _NBUF * k)
    src = jnp.pad(g[0], (0, e_pad - n_edges))
    dst = jnp.pad(g[1], (0, e_pad - n_edges))
    w_p = jnp.pad(edge_weight, (0, e_pad - n_edges))

    q_pair = e_pad // (ns * k)
    q0 = (int(q_pair * 0.72) // _NBUF) * _NBUF
    q1 = q_pair - q0
    assert q1 % _NBUF == 0

    sp_arr = (src | (dst << 16)).reshape(-1, k)
    w_arr = w_p.reshape(-1, k)
    extra = max(q0, q1) - min(q0, q1)
    sp_arr = jnp.pad(sp_arr, ((0, extra), (0, 0)))
    w_arr = jnp.pad(w_arr, ((0, extra), (0, 0)))

    n_pad = ((n_nodes + 8 * ns - 1) // (8 * ns)) * (8 * ns)

    scatter_h = _make_sc_scatter(n_pad, e_pad, d_h, q0, q1)
    scatter_o = _make_sc_scatter(n_pad, e_pad, d_out, q0, q1)

    p1 = _tc_matmul(features, W1)
    a1 = scatter_h(p1, sp_arr, w_arr)
    p2 = _tc_fused(a1[0, :n_nodes], a1[1, :n_nodes], b1, W2)
    a2 = scatter_h(p2, sp_arr, w_arr)
    p3 = _tc_fused(a2[0, :n_nodes], a2[1, :n_nodes], b2, W3p)
    a3 = scatter_o(p3, sp_arr, w_arr)
    out = _tc_add(a3[0, :n_nodes], a3[1, :n_nodes], b3p)
    return out[:, :n_classes]


# speedup vs baseline: 1.0060x; 1.0060x over previous
import functools

import jax
import jax.numpy as jnp
from jax import lax
from jax.experimental import pallas as pl
from jax.experimental.pallas import tpu as pltpu
from jax.experimental.pallas import tpu_sc as plsc


def _mm_body(x_ref, w_ref, o_ref):
    o_ref[...] = jnp.dot(x_ref[...], w_ref[...],
                         preferred_element_type=jnp.float32)


def _fused_body(a_ref, b_ref, bias_ref, w_ref, o_ref):
    h = jnp.maximum(a_ref[...] + b_ref[...] + bias_ref[...], 0.0)
    o_ref[...] = jnp.dot(h, w_ref[...], preferred_element_type=jnp.float32)


def _add_body(a_ref, b_ref, bias_ref, o_ref):
    o_ref[...] = a_ref[...] + b_ref[...] + bias_ref[...]


def _tc_matmul(x, w, block_rows=2000):
    n, d_in = x.shape
    d_out = w.shape[1]
    return pl.pallas_call(
        _mm_body,
        grid=(n // block_rows,),
        in_specs=[
            pl.BlockSpec((block_rows, d_in), lambda i: (i, 0)),
            pl.BlockSpec((d_in, d_out), lambda i: (0, 0)),
        ],
        out_specs=pl.BlockSpec((block_rows, d_out), lambda i: (i, 0)),
        out_shape=jax.ShapeDtypeStruct((n, d_out), jnp.float32),
    )(x, w)


def _tc_fused(a, b, bias, w, block_rows=2000):
    n, d_in = a.shape
    d_out = w.shape[1]
    bias2 = bias.reshape(1, d_in)
    return pl.pallas_call(
        _fused_body,
        grid=(n // block_rows,),
        in_specs=[
            pl.BlockSpec((block_rows, d_in), lambda i: (i, 0)),
            pl.BlockSpec((block_rows, d_in), lambda i: (i, 0)),
            pl.BlockSpec((1, d_in), lambda i: (0, 0)),
            pl.BlockSpec((d_in, d_out), lambda i: (0, 0)),
        ],
        out_specs=pl.BlockSpec((block_rows, d_out), lambda i: (i, 0)),
        out_shape=jax.ShapeDtypeStruct((n, d_out), jnp.float32),
    )(a, b, bias2, w)


def _tc_add(a, b, bias, block_rows=2000):
    n, d = a.shape
    bias2 = bias.reshape(1, d)
    return pl.pallas_call(
        _add_body,
        grid=(n // block_rows,),
        in_specs=[
            pl.BlockSpec((block_rows, d), lambda i: (i, 0)),
            pl.BlockSpec((block_rows, d), lambda i: (i, 0)),
            pl.BlockSpec((1, d), lambda i: (0, 0)),
        ],
        out_specs=pl.BlockSpec((block_rows, d), lambda i: (i, 0)),
        out_shape=jax.ShapeDtypeStruct((n, d), jnp.float32),
    )(a, b, bias2)


_SC_K = 32
_NBUF = 4


@functools.lru_cache(maxsize=None)
def _make_sc_scatter(n_nodes, n_edges, d, q0, q1):
    info = plsc.get_sparse_core_info()
    nc, ns, lanes = info.num_cores, info.num_subcores, info.num_lanes
    k = _SC_K
    nb = _NBUF
    qmax = max(q0, q1)
    rows_per_tile = n_nodes // ns
    assert n_edges == ns * (q0 + q1) * k and q0 % nb == 0 and q1 % nb == 0
    assert n_nodes % (8 * ns) == 0 and d % lanes == 0
    assert rows_per_tile % 8 == 0

    mesh = plsc.VectorSubcoreMesh(core_axis_name="c", subcore_axis_name="s")
    _bcast_dn = lax.GatherDimensionNumbers(
        offset_dims=(), collapsed_slice_dims=(0,), start_index_map=(0,))

    @functools.partial(
        pl.kernel,
        mesh=mesh,
        compiler_params=pltpu.CompilerParams(use_tc_tiling_on_sc=False),
        out_type=jax.ShapeDtypeStruct((nc, n_nodes, d), jnp.float32),
        scratch_types=[
            pltpu.VMEM_SHARED((n_nodes, d), jnp.float32),
            pltpu.VMEM((qmax, k), jnp.int32),
            pltpu.VMEM((qmax, k), jnp.float32),
            pltpu.VMEM((nb, 2, k), jnp.int32),
            pltpu.VMEM((nb, k, d), jnp.float32),
            pltpu.SemaphoreType.DMA((nb,)),
            pltpu.SemaphoreType.DMA((nb,)),
        ],
    )
    def sc_scatter(p_hbm, sp_hbm, w_hbm, out_hbm,
                   agg, sp_v, w_v, idx_v, rows_v, gsem, ssem):
        cid = lax.axis_index("c")
        sid = lax.axis_index("s")

        my_rows = pl.ds(sid * rows_per_tile, rows_per_tile)
        nq = jnp.where(cid == 0, q0, q1)
        n_quads_t = nq // nb
        cbase = jnp.where(cid == 0, sid * q0, ns * q0 + sid * q1)
        pre_sp = pltpu.make_async_copy(
            sp_hbm.at[pl.ds(cbase, qmax)], sp_v, gsem.at[0])
        pre_w = pltpu.make_async_copy(
            w_hbm.at[pl.ds(cbase, qmax)], w_v, gsem.at[1])
        pre_sp.start()
        pre_w.start()

        zv = jnp.zeros((lanes,), jnp.float32)

        def zrow_body(r, carry):
            for j in range(d // lanes):
                rows_v[0, r, pl.ds(j * lanes, lanes)] = zv
            return carry

        lax.fori_loop(0, k, zrow_body, 0)
        for t in range(rows_per_tile // k):
            pltpu.sync_copy(
                rows_v.at[0],
                agg.at[pl.ds(sid * rows_per_tile + t * k, k)])
        rem = rows_per_tile % k
        if rem:
            pltpu.sync_copy(
                rows_v.at[0, pl.ds(0, rem)],
                agg.at[pl.ds(sid * rows_per_tile + rows_per_tile - rem, rem)])
        pre_sp.wait()
        pre_w.wait()
        plsc.subcore_barrier()

        def unpack(c, b):
            for t in range(k // lanes):
                sl = pl.ds(t * lanes, lanes)
                v = sp_v[c, sl]
                idx_v[b, 0, sl] = v & 0xFFFF
                idx_v[b, 1, sl] = (v >> 16) & 0xFFFF

        def gather(b, c):
            return pltpu.make_async_copy(
                p_hbm.at[idx_v.at[b, 0]], rows_v.at[b], gsem.at[b])

        def scatter(b, c):
            return pltpu.make_async_copy(
                rows_v.at[b], agg.at[idx_v.at[b, 1]], ssem.at[b])

        unpack(0, 0)
        unpack(1, 1)
        gather(0, 0).start()
        gather(1, 1).start()

        def quad_body(cc, carry):
            for b in range(nb):
                c = nb * cc + b
                b2 = (b + 2) % nb
                gather(b, c).wait()
                if b < 2:
                    @pl.when(cc > 0)
                    def _():
                        scatter(b2, c).wait()

                    unpack(c + 2, b2)
                    gather(b2, c + 2).start()
                else:
                    scatter(b2, c).wait()

                    @pl.when(cc < n_quads_t - 1)
                    def _():
                        unpack(c + 2, b2)
                        gather(b2, c + 2).start()

                for t in range(k // lanes):
                    wv16 = w_v[c, pl.ds(t * lanes, lanes)]
                    for l in range(lanes):
                        wb = lax.gather(
                            wv16, jnp.full((lanes, 1), l, jnp.int32),
                            _bcast_dn, (1,),
                            mode=lax.GatherScatterMode.PROMISE_IN_BOUNDS)
                        i = t * lanes + l
                        for j in range(d // lanes):
                            sl = pl.ds(j * lanes, lanes)
                            rows_v[b, i, sl] = rows_v[b, i, sl] * wb
                scatter(b, c).start(add=True)
            return carry

        lax.fori_loop(0, n_quads_t, quad_body, 0)
        scatter(2, 0).wait()
        scatter(3, 0).wait()
        plsc.subcore_barrier()
        pltpu.sync_copy(agg.at[my_rows], out_hbm.at[cid, my_rows])

    return sc_scatter


def kernel(features, g, edge_weight, W1, b1, W2, b2, W3, b3):
    n_nodes, d_in = features.shape
    n_edges = g.shape[1]
    d_h = W1.shape[1]
    n_classes = W3.shape[1]
    lanes = 16
    d_out = ((n_classes + lanes - 1) // lanes) * lanes

    W3p = jnp.pad(W3, ((0, 0), (0, d_out - n_classes)))
    b3p = jnp.pad(b3, (0, d_out - n_classes))

    info = plsc.get_sparse_core_info()
    ns, nc = info.num_subcores, info.num_cores
    nw = nc * ns
    k = _SC_K
    e_pad = -(-n_edges // (nw * _NBUF * k)) * (nw * _NBUF * k)
    src = jnp.pad(g[0], (0, e_pad - n_edges))
    dst = jnp.pad(g[1], (0, e_pad - n_edges))
    w_p = jnp.pad(edge_weight, (0, e_pad - n_edges))

    q_pair = e_pad // (ns * k)
    q0 = (int(q_pair * 0.68) // _NBUF) * _NBUF
    q1 = q_pair - q0
    assert q1 % _NBUF == 0

    sp_arr = (src | (dst << 16)).reshape(-1, k)
    w_arr = w_p.reshape(-1, k)
    extra = max(q0, q1) - min(q0, q1)
    sp_arr = jnp.pad(sp_arr, ((0, extra), (0, 0)))
    w_arr = jnp.pad(w_arr, ((0, extra), (0, 0)))

    n_pad = ((n_nodes + 8 * ns - 1) // (8 * ns)) * (8 * ns)

    scatter_h = _make_sc_scatter(n_pad, e_pad, d_h, q0, q1)
    scatter_o = _make_sc_scatter(n_pad, e_pad, d_out, q0, q1)

    p1 = _tc_matmul(features, W1)
    a1 = scatter_h(p1, sp_arr, w_arr)
    p2 = _tc_fused(a1[0, :n_nodes], a1[1, :n_nodes], b1, W2)
    a2 = scatter_h(p2, sp_arr, w_arr)
    p3 = _tc_fused(a2[0, :n_nodes], a2[1, :n_nodes], b2, W3p)
    a3 = scatter_o(p3, sp_arr, w_arr)
    out = _tc_add(a3[0, :n_nodes], a3[1, :n_nodes], b3p)
    return out[:, :n_classes]


# speedup vs baseline: 1.0121x; 1.0060x over previous
import functools

import jax
import jax.numpy as jnp
from jax import lax
from jax.experimental import pallas as pl
from jax.experimental.pallas import tpu as pltpu
from jax.experimental.pallas import tpu_sc as plsc


def _mm_body(x_ref, w_ref, o_ref):
    o_ref[...] = jnp.dot(x_ref[...], w_ref[...],
                         preferred_element_type=jnp.float32)


def _fused_body(a_ref, b_ref, bias_ref, w_ref, o_ref):
    h = jnp.maximum(a_ref[...] + b_ref[...] + bias_ref[...], 0.0)
    o_ref[...] = jnp.dot(h, w_ref[...], preferred_element_type=jnp.float32)


def _add_body(a_ref, b_ref, bias_ref, o_ref):
    o_ref[...] = a_ref[...] + b_ref[...] + bias_ref[...]


def _tc_matmul(x, w, block_rows=10000):
    n, d_in = x.shape
    d_out = w.shape[1]
    return pl.pallas_call(
        _mm_body,
        grid=(n // block_rows,),
        in_specs=[
            pl.BlockSpec((block_rows, d_in), lambda i: (i, 0)),
            pl.BlockSpec((d_in, d_out), lambda i: (0, 0)),
        ],
        out_specs=pl.BlockSpec((block_rows, d_out), lambda i: (i, 0)),
        out_shape=jax.ShapeDtypeStruct((n, d_out), jnp.float32),
    )(x, w)


def _tc_fused(a, b, bias, w, block_rows=10000):
    n, d_in = a.shape
    d_out = w.shape[1]
    bias2 = bias.reshape(1, d_in)
    return pl.pallas_call(
        _fused_body,
        grid=(n // block_rows,),
        in_specs=[
            pl.BlockSpec((block_rows, d_in), lambda i: (i, 0)),
            pl.BlockSpec((block_rows, d_in), lambda i: (i, 0)),
            pl.BlockSpec((1, d_in), lambda i: (0, 0)),
            pl.BlockSpec((d_in, d_out), lambda i: (0, 0)),
        ],
        out_specs=pl.BlockSpec((block_rows, d_out), lambda i: (i, 0)),
        out_shape=jax.ShapeDtypeStruct((n, d_out), jnp.float32),
    )(a, b, bias2, w)


def _tc_add(a, b, bias, block_rows=10000):
    n, d = a.shape
    bias2 = bias.reshape(1, d)
    return pl.pallas_call(
        _add_body,
        grid=(n // block_rows,),
        in_specs=[
            pl.BlockSpec((block_rows, d), lambda i: (i, 0)),
            pl.BlockSpec((block_rows, d), lambda i: (i, 0)),
            pl.BlockSpec((1, d), lambda i: (0, 0)),
        ],
        out_specs=pl.BlockSpec((block_rows, d), lambda i: (i, 0)),
        out_shape=jax.ShapeDtypeStruct((n, d), jnp.float32),
    )(a, b, bias2)


_SC_K = 32
_NBUF = 4


@functools.lru_cache(maxsize=None)
def _make_sc_scatter(n_nodes, n_edges, d, q0, q1):
    info = plsc.get_sparse_core_info()
    nc, ns, lanes = info.num_cores, info.num_subcores, info.num_lanes
    k = _SC_K
    nb = _NBUF
    qmax = max(q0, q1)
    rows_per_tile = n_nodes // ns
    assert n_edges == ns * (q0 + q1) * k and q0 % nb == 0 and q1 % nb == 0
    assert n_nodes % (8 * ns) == 0 and d % lanes == 0
    assert rows_per_tile % 8 == 0

    mesh = plsc.VectorSubcoreMesh(core_axis_name="c", subcore_axis_name="s")
    _bcast_dn = lax.GatherDimensionNumbers(
        offset_dims=(), collapsed_slice_dims=(0,), start_index_map=(0,))

    @functools.partial(
        pl.kernel,
        mesh=mesh,
        compiler_params=pltpu.CompilerParams(use_tc_tiling_on_sc=False),
        out_type=jax.ShapeDtypeStruct((nc, n_nodes, d), jnp.float32),
        scratch_types=[
            pltpu.VMEM_SHARED((n_nodes, d), jnp.float32),
            pltpu.VMEM((qmax, k), jnp.int32),
            pltpu.VMEM((qmax, k), jnp.float32),
            pltpu.VMEM((nb, 2, k), jnp.int32),
            pltpu.VMEM((nb, k, d), jnp.float32),
            pltpu.SemaphoreType.DMA((nb,)),
            pltpu.SemaphoreType.DMA((nb,)),
        ],
    )
    def sc_scatter(p_hbm, sp_hbm, w_hbm, out_hbm,
                   agg, sp_v, w_v, idx_v, rows_v, gsem, ssem):
        cid = lax.axis_index("c")
        sid = lax.axis_index("s")

        my_rows = pl.ds(sid * rows_per_tile, rows_per_tile)
        nq = jnp.where(cid == 0, q0, q1)
        n_quads_t = nq // nb
        cbase = jnp.where(cid == 0, sid * q0, ns * q0 + sid * q1)
        pre_sp = pltpu.make_async_copy(
            sp_hbm.at[pl.ds(cbase, qmax)], sp_v, gsem.at[0])
        pre_w = pltpu.make_async_copy(
            w_hbm.at[pl.ds(cbase, qmax)], w_v, gsem.at[1])
        pre_sp.start()
        pre_w.start()

        zv = jnp.zeros((lanes,), jnp.float32)

        def zrow_body(r, carry):
            for j in range(d // lanes):
                rows_v[0, r, pl.ds(j * lanes, lanes)] = zv
            return carry

        lax.fori_loop(0, k, zrow_body, 0)
        for t in range(rows_per_tile // k):
            pltpu.sync_copy(
                rows_v.at[0],
                agg.at[pl.ds(sid * rows_per_tile + t * k, k)])
        rem = rows_per_tile % k
        if rem:
            pltpu.sync_copy(
                rows_v.at[0, pl.ds(0, rem)],
                agg.at[pl.ds(sid * rows_per_tile + rows_per_tile - rem, rem)])
        pre_sp.wait()
        pre_w.wait()
        plsc.subcore_barrier()

        def unpack(c, b):
            for t in range(k // lanes):
                sl = pl.ds(t * lanes, lanes)
                v = sp_v[c, sl]
                idx_v[b, 0, sl] = v & 0xFFFF
                idx_v[b, 1, sl] = (v >> 16) & 0xFFFF

        def gather(b, c):
            return pltpu.make_async_copy(
                p_hbm.at[idx_v.at[b, 0]], rows_v.at[b], gsem.at[b])

        def scatter(b, c):
            return pltpu.make_async_copy(
                rows_v.at[b], agg.at[idx_v.at[b, 1]], ssem.at[b])

        unpack(0, 0)
        unpack(1, 1)
        gather(0, 0).start()
        gather(1, 1).start()

        def quad_body(cc, carry):
            for b in range(nb):
                c = nb * cc + b
                b2 = (b + 2) % nb
                gather(b, c).wait()
                if b < 2:
                    @pl.when(cc > 0)
                    def _():
                        scatter(b2, c).wait()

                    unpack(c + 2, b2)
                    gather(b2, c + 2).start()
                else:
                    scatter(b2, c).wait()

                    @pl.when(cc < n_quads_t - 1)
                    def _():
                        unpack(c + 2, b2)
                        gather(b2, c + 2).start()

                for t in range(k // lanes):
                    wv16 = w_v[c, pl.ds(t * lanes, lanes)]
                    for l in range(lanes):
                        wb = lax.gather(
                            wv16, jnp.full((lanes, 1), l, jnp.int32),
                            _bcast_dn, (1,),
                            mode=lax.GatherScatterMode.PROMISE_IN_BOUNDS)
                        i = t * lanes + l
                        for j in range(d // lanes):
                            sl = pl.ds(j * lanes, lanes)
                            rows_v[b, i, sl] = rows_v[b, i, sl] * wb
                scatter(b, c).start(add=True)
            return carry

        lax.fori_loop(0, n_quads_t, quad_body, 0)
        scatter(2, 0).wait()
        scatter(3, 0).wait()
        plsc.subcore_barrier()
        pltpu.sync_copy(agg.at[my_rows], out_hbm.at[cid, my_rows])

    return sc_scatter


def kernel(features, g, edge_weight, W1, b1, W2, b2, W3, b3):
    n_nodes, d_in = features.shape
    n_edges = g.shape[1]
    d_h = W1.shape[1]
    n_classes = W3.shape[1]
    lanes = 16
    d_out = ((n_classes + lanes - 1) // lanes) * lanes

    W3p = jnp.pad(W3, ((0, 0), (0, d_out - n_classes)))
    b3p = jnp.pad(b3, (0, d_out - n_classes))

    info = plsc.get_sparse_core_info()
    ns, nc = info.num_subcores, info.num_cores
    nw = nc * ns
    k = _SC_K
    e_pad = -(-n_edges // (nw * _NBUF * k)) * (nw * _NBUF * k)
    src = jnp.pad(g[0], (0, e_pad - n_edges))
    dst = jnp.pad(g[1], (0, e_pad - n_edges))
    w_p = jnp.pad(edge_weight, (0, e_pad - n_edges))

    q_pair = e_pad // (ns * k)
    q0 = (int(q_pair * 0.68) // _NBUF) * _NBUF
    q1 = q_pair - q0
    assert q1 % _NBUF == 0

    sp_arr = (src | (dst << 16)).reshape(-1, k)
    w_arr = w_p.reshape(-1, k)
    extra = max(q0, q1) - min(q0, q1)
    sp_arr = jnp.pad(sp_arr, ((0, extra), (0, 0)))
    w_arr = jnp.pad(w_arr, ((0, extra), (0, 0)))

    n_pad = ((n_nodes + 8 * ns - 1) // (8 * ns)) * (8 * ns)

    scatter_h = _make_sc_scatter(n_pad, e_pad, d_h, q0, q1)
    scatter_o = _make_sc_scatter(n_pad, e_pad, d_out, q0, q1)

    p1 = _tc_matmul(features, W1)
    a1 = scatter_h(p1, sp_arr, w_arr)
    p2 = _tc_fused(a1[0, :n_nodes], a1[1, :n_nodes], b1, W2)
    a2 = scatter_h(p2, sp_arr, w_arr)
    p3 = _tc_fused(a2[0, :n_nodes], a2[1, :n_nodes], b2, W3p)
    a3 = scatter_o(p3, sp_arr, w_arr)
    out = _tc_add(a3[0, :n_nodes], a3[1, :n_nodes], b3p)
    return out[:, :n_classes]
